# baseline (device time: 23720 ns/iter reference)
import jax
import jax.numpy as jnp
from jax import lax
from jax.experimental import pallas as pl
from jax.experimental.pallas import tpu as pltpu

N_DEV = 32


def kernel(x, router_W, route_idx, expert_W):
    n, d = x.shape
    e_per, _, h = expert_W.shape
    rows = n // N_DEV

    def body(x_ref, route_ref, w_ref, out_ref,
             rbuf, acc, rs_send, rs_recv, ag_send, ag_recv):
        m = lax.axis_index("i")

        barrier_sem = pltpu.get_barrier_semaphore()
        for j in range(N_DEV):
            @pl.when(m != j)
            def _(j=j):
                pl.semaphore_signal(
                    barrier_sem, inc=1,
                    device_id=(j,), device_id_type=pl.DeviceIdType.MESH,
                )
        pl.semaphore_wait(barrier_sem, N_DEV - 1)

        routes = route_ref[:, :]
        contrib = jnp.zeros((n, h), jnp.float32)
        for e in range(e_per):
            eid = m * e_per + e
            mask = (routes == eid).astype(jnp.float32)
            contrib = contrib + jnp.dot(
                x_ref[:, :] * mask, w_ref[e],
                preferred_element_type=jnp.float32,
            )
        out_ref[:, :] = contrib

        for j in range(N_DEV):
            @pl.when(m == j)
            def _(j=j):
                rbuf[j] = out_ref[pl.ds(j * rows, rows), :]

            @pl.when(m != j)
            def _(j=j):
                pltpu.make_async_remote_copy(
                    src_ref=out_ref.at[pl.ds(j * rows, rows), :],
                    dst_ref=rbuf.at[m],
                    send_sem=rs_send.at[j],
                    recv_sem=rs_recv.at[m],
                    device_id=(j,),
                    device_id_type=pl.DeviceIdType.MESH,
                ).start()
        for j in range(N_DEV):
            @pl.when(m != j)
            def _(j=j):
                pltpu.make_async_remote_copy(
                    src_ref=rbuf.at[j],
                    dst_ref=rbuf.at[j],
                    send_sem=rs_send.at[j],
                    recv_sem=rs_recv.at[j],
                    device_id=(m,),
                    device_id_type=pl.DeviceIdType.MESH,
                ).wait_recv()
        acc[:, :] = jnp.sum(rbuf[:, :, :], axis=0)

        for j in range(N_DEV):
            @pl.when(m == j)
            def _(j=j):
                out_ref[pl.ds(j * rows, rows), :] = acc[:, :]

            @pl.when(m != j)
            def _(j=j):
                pltpu.make_async_remote_copy(
                    src_ref=acc,
                    dst_ref=out_ref.at[pl.ds(m * rows, rows), :],
                    send_sem=ag_send.at[j],
                    recv_sem=ag_recv.at[m],
                    device_id=(j,),
                    device_id_type=pl.DeviceIdType.MESH,
                ).start()
        for j in range(N_DEV):
            @pl.when(m != j)
            def _(j=j):
                pltpu.make_async_remote_copy(
                    src_ref=out_ref.at[pl.ds(j * rows, rows), :],
                    dst_ref=out_ref.at[pl.ds(j * rows, rows), :],
                    send_sem=ag_send.at[j],
                    recv_sem=ag_recv.at[j],
                    device_id=(m,),
                    device_id_type=pl.DeviceIdType.MESH,
                ).wait_recv()

        for j in range(N_DEV):
            @pl.when(m != j)
            def _(j=j):
                pltpu.make_async_remote_copy(
                    src_ref=out_ref.at[pl.ds(j * rows, rows), :],
                    dst_ref=rbuf.at[j],
                    send_sem=rs_send.at[j],
                    recv_sem=rs_recv.at[j],
                    device_id=(m,),
                    device_id_type=pl.DeviceIdType.MESH,
                ).wait_send()
                pltpu.make_async_remote_copy(
                    src_ref=acc,
                    dst_ref=out_ref.at[pl.ds(j * rows, rows), :],
                    send_sem=ag_send.at[j],
                    recv_sem=ag_recv.at[j],
                    device_id=(m,),
                    device_id_type=pl.DeviceIdType.MESH,
                ).wait_send()

    return pl.pallas_call(
        body,
        out_shape=jax.ShapeDtypeStruct((n, h), jnp.float32),
        in_specs=[
            pl.BlockSpec(memory_space=pltpu.VMEM),
            pl.BlockSpec(memory_space=pltpu.VMEM),
            pl.BlockSpec(memory_space=pltpu.VMEM),
        ],
        out_specs=pl.BlockSpec(memory_space=pltpu.VMEM),
        scratch_shapes=[
            pltpu.VMEM((N_DEV, rows, h), jnp.float32),
            pltpu.VMEM((rows, h), jnp.float32),
            pltpu.SemaphoreType.DMA((N_DEV,)),
            pltpu.SemaphoreType.DMA((N_DEV,)),
            pltpu.SemaphoreType.DMA((N_DEV,)),
            pltpu.SemaphoreType.DMA((N_DEV,)),
        ],
        compiler_params=pltpu.CompilerParams(collective_id=0),
    )(x, route_idx, expert_W)
